# baseline (speedup 1.0000x reference)
import functools

import jax
import jax.numpy as jnp
from jax import lax
from jax.experimental import pallas as pl
from jax.experimental.pallas import tpu as pltpu
from jax.experimental.pallas import tpu_sc as plsc

BATCH = 16384
D = 64
SUB = 8
LANES = 16
N_CORES = 2
N_SUBCORES = 16
N_WORKERS = N_CORES * N_SUBCORES
B_PER_W = BATCH // N_WORKERS
CHUNK = 128
N_CHUNKS = B_PER_W // CHUNK
N_GROUPS = CHUNK // LANES


@functools.partial(
    pl.kernel,
    out_type=jax.ShapeDtypeStruct((BATCH,), jnp.float32),
    mesh=plsc.VectorSubcoreMesh(core_axis_name="c", subcore_axis_name="s"),
    compiler_params=pltpu.CompilerParams(needs_layout_passes=False),
    scratch_types=[
        pltpu.VMEM((B_PER_W,), jnp.int32),
        pltpu.VMEM((B_PER_W,), jnp.int32),
        pltpu.VMEM((CHUNK, D), jnp.float32),
        pltpu.VMEM((CHUNK, D), jnp.float32),
        pltpu.VMEM((CHUNK, D), jnp.float32),
        pltpu.VMEM((CHUNK, D), jnp.float32),
        pltpu.VMEM((B_PER_W,), jnp.float32),
        pltpu.SemaphoreType.DMA,
        pltpu.SemaphoreType.DMA,
        pltpu.SemaphoreType.DMA,
        pltpu.SemaphoreType.DMA,
    ],
)
def _mf_kernel(user_hbm, item_hbm, tu_hbm, ti_hbm, out_hbm,
               idx_u, idx_i, ru0, ri0, ru1, ri1, out_v,
               su0, si0, su1, si1):
    wid = lax.axis_index("s") * N_CORES + lax.axis_index("c")
    base = wid * B_PER_W

    pltpu.sync_copy(user_hbm.at[pl.ds(base, B_PER_W)], idx_u)
    pltpu.sync_copy(item_hbm.at[pl.ds(base, B_PER_W)], idx_i)

    lane = lax.iota(jnp.int32, LANES)
    last_lane = lane == (LANES - 1)

    bufs = ((ru0, ri0, su0, si0), (ru1, ri1, su1, si1))

    def fire_chunk(c):
        rows_u, rows_i, sem_u, sem_i = bufs[c % 2]

        def fire(g, _):
            b0 = c * CHUNK + g * LANES
            vu = idx_u[pl.ds(b0, LANES)]
            vi = idx_i[pl.ds(b0, LANES)]
            for j in range(LANES):
                row = g * LANES + j
                ru = vu[j]
                pltpu.async_copy(
                    tu_hbm.at[ru >> 3, pl.ds(ru & 7, 1), :],
                    rows_u.at[pl.ds(row, 1), :], sem_u)
                ri = vi[j]
                pltpu.async_copy(
                    ti_hbm.at[ri >> 3, pl.ds(ri & 7, 1), :],
                    rows_i.at[pl.ds(row, 1), :], sem_i)
            return _

        lax.fori_loop(0, N_GROUPS, fire, None)

    def drain_chunk(c):
        rows_u, rows_i, sem_u, sem_i = bufs[c % 2]
        for start in range(0, CHUNK, SUB):
            pltpu.make_async_copy(tu_hbm.at[0],
                                  rows_u.at[pl.ds(start, SUB), :],
                                  sem_u).wait()
            pltpu.make_async_copy(ti_hbm.at[0],
                                  rows_i.at[pl.ds(start, SUB), :],
                                  sem_i).wait()

    def compute_chunk(c):
        rows_u, rows_i, _, _ = bufs[c % 2]

        def group(g, _):
            b0 = c * CHUNK + g * LANES
            for j in range(LANES):
                row = g * LANES + j
                acc = None
                for k in range(D // LANES):
                    u = rows_u[row, pl.ds(k * LANES, LANES)]
                    v = rows_i[row, pl.ds(k * LANES, LANES)]
                    acc = u * v if acc is None else acc + u * v
                total = plsc.cumsum(acc)
                plsc.store_scatter(out_v,
                                   [jnp.full((LANES,), b0 + j, jnp.int32)],
                                   total, mask=last_lane)
            return _

        lax.fori_loop(0, N_GROUPS, group, None)

    fire_chunk(0)
    for c in range(N_CHUNKS):
        if c + 1 < N_CHUNKS:
            fire_chunk(c + 1)
        drain_chunk(c)
        compute_chunk(c)

    pltpu.sync_copy(out_v, out_hbm.at[pl.ds(base, B_PER_W)])


def kernel(user, item, user_factors, item_factors):
    user = user.astype(jnp.int32)
    item = item.astype(jnp.int32)
    tu = user_factors.reshape(user_factors.shape[0] // SUB, SUB, D)
    ti = item_factors.reshape(item_factors.shape[0] // SUB, SUB, D)
    return _mf_kernel(user, item, tu, ti)
